# baseline (speedup 1.0000x reference)
import functools

import jax
import jax.numpy as jnp
import numpy as np
from jax import lax
from jax.experimental import pallas as pl
from jax.experimental.pallas import tpu as pltpu
from jax.experimental.pallas import tpu_sc as plsc

N = 10000
D = 128
HD = 64
B = 16
L = 625
LP = 640
NUM_LAYERS = 4

NTILES = 32
CHUNK = 128
NROWS = 10240
DUMMY = N
SUBROWS = NROWS // 16
ECOUNTS = (80000, 80000, 80000, 80000, 20000, 20000)
def _mult4(x):
    return -(-x // 4) * 4


CPT = tuple(_mult4(-(-e // (NTILES * CHUNK))) for e in ECOUNTS)
OFF = tuple(int(np.cumsum((0,) + CPT)[i]) for i in range(6))
TOT = sum(CPT)

_DN_T = (((1,), (1,)), ((), ()))
_DN_N = (((1,), (0,)), ((), ()))


def _attn_body(x_ref, wq_ref, wk_ref, wv_ref, oa_ref, ob_ref):
    x = x_ref[0]
    q = lax.dot_general(x, wq_ref[...], _DN_T, preferred_element_type=jnp.float32)
    k = lax.dot_general(x, wk_ref[...], _DN_T, preferred_element_type=jnp.float32)
    v = lax.dot_general(x, wv_ref[...], _DN_T, preferred_element_type=jnp.float32)
    e = lax.dot_general(q, k, _DN_T, preferred_element_type=jnp.float32)
    e = e * jnp.float32(1.0 / np.sqrt(D))
    col = lax.broadcasted_iota(jnp.int32, (LP, LP), 1)
    e = jnp.where(col < L, e, jnp.float32(-1e30))
    m = jnp.max(e, axis=-1, keepdims=True)
    p = jnp.exp(e - m)
    p = p / jnp.sum(p, axis=-1, keepdims=True)
    att = lax.dot_general(p, v, _DN_N, preferred_element_type=jnp.float32)
    o = x + att
    oa_ref[0] = o[:, :HD]
    ob_ref[0] = o[:, HD:]


def _attention(feats_p, wq, wk, wv):
    return pl.pallas_call(
        _attn_body,
        grid=(B,),
        in_specs=[
            pl.BlockSpec((1, LP, D), lambda b: (b, 0, 0)),
            pl.BlockSpec((D, D), lambda b: (0, 0)),
            pl.BlockSpec((D, D), lambda b: (0, 0)),
            pl.BlockSpec((D, D), lambda b: (0, 0)),
        ],
        out_specs=[
            pl.BlockSpec((1, LP, HD), lambda b: (b, 0, 0)),
            pl.BlockSpec((1, LP, HD), lambda b: (b, 0, 0)),
        ],
        out_shape=[
            jax.ShapeDtypeStruct((B, LP, HD), jnp.float32),
            jax.ShapeDtypeStruct((B, LP, HD), jnp.float32),
        ],
    )(feats_p, wq, wk, wv)


_MESH = plsc.VectorSubcoreMesh(core_axis_name="c", subcore_axis_name="s")


@functools.partial(
    pl.kernel,
    out_type=jax.ShapeDtypeStruct((2, 2, 6, NROWS, HD), jnp.float32),
    mesh=_MESH,
    compiler_params=pltpu.CompilerParams(use_tc_tiling_on_sc=False),
    scratch_types=[
        pltpu.VMEM((TOT, CHUNK), jnp.int32),
        pltpu.VMEM((TOT, CHUNK), jnp.int32),
        pltpu.VMEM((CHUNK, HD), jnp.float32),
        pltpu.VMEM((CHUNK, HD), jnp.float32),
        pltpu.VMEM((CHUNK, HD), jnp.float32),
        pltpu.VMEM((CHUNK, HD), jnp.float32),
        pltpu.VMEM((CHUNK, HD), jnp.float32),
        pltpu.SemaphoreType.DMA,
        pltpu.SemaphoreType.DMA,
        pltpu.SemaphoreType.DMA,
        pltpu.SemaphoreType.DMA,
        pltpu.SemaphoreType.DMA,
        pltpu.SemaphoreType.DMA,
        pltpu.SemaphoreType.DMA,
        pltpu.SemaphoreType.DMA,
        pltpu.VMEM_SHARED((NROWS, HD), jnp.float32),
    ],
)
def _sc_agg(fa_hbm, fb_hbm, u_hbm, v_hbm, out_hbm, u_v, v_v, b0, b1, b2, b3,
            z_v, g0, g1, g2, g3, s0, s1, s2, s3, agg_sh):
    bufs = (b0, b1, b2, b3)
    gs = (g0, g1, g2, g3)
    ss = (s0, s1, s2, s3)
    cid = lax.axis_index("c")
    sid = lax.axis_index("s")
    wid = cid * 16 + sid
    pltpu.sync_copy(u_hbm.at[wid], u_v)
    pltpu.sync_copy(v_hbm.at[wid], v_v)

    def zero_body(i, _):
        z_v[i // 4, pl.ds((i % 4) * 16, 16)] = jnp.zeros((16,), jnp.float32)
        return 0

    lax.fori_loop(0, CHUNK * 4, zero_body, 0)
    row0 = sid * SUBROWS

    def prologue(f_hbm, t):
        for k in range(4):
            pltpu.async_copy(f_hbm.at[v_v.at[OFF[t] + k]], bufs[k], gs[k])

    for d, f_hbm in enumerate((fa_hbm, fb_hbm)):
        if d == 0:
            prologue(f_hbm, 0)
        for t in range(6):
            if t % 2 == 0:
                for j in range(SUBROWS // CHUNK):
                    pltpu.sync_copy(
                        z_v, agg_sh.at[pl.ds(row0 + j * CHUNK, CHUNK)])
                plsc.subcore_barrier()
            lo = OFF[t]
            nq = CPT[t] // 4

            def quad_body(i, _):
                base = lo + 4 * i
                for k in range(4):
                    c = base + k
                    pltpu.make_async_copy(
                        f_hbm.at[v_v.at[c]], bufs[k], gs[k]).wait()
                    pltpu.async_copy(
                        bufs[k], agg_sh.at[u_v.at[c]], ss[k], add=True)
                for k in range(4):
                    c = base + k
                    pltpu.make_async_copy(
                        bufs[k], agg_sh.at[u_v.at[c]], ss[k]).wait()

                    @pl.when(i < nq - 1)
                    def _(k=k, c=c):
                        pltpu.async_copy(
                            f_hbm.at[v_v.at[c + 4]], bufs[k], gs[k])
                return 0

            lax.fori_loop(0, nq, quad_body, 0)
            if t < 5:
                prologue(f_hbm, t + 1)
            elif d == 0:
                prologue(fb_hbm, 0)
            plsc.subcore_barrier()
            pltpu.sync_copy(
                agg_sh.at[pl.ds(row0, SUBROWS)],
                out_hbm.at[d, cid, t, pl.ds(row0, SUBROWS)],
            )
            plsc.subcore_barrier()


HN = N // 2
RB = 1000


def _layer_body(fa_ref, fb_ref, agg_ref, wc_ref, we_ref, wc2_ref, g1_ref,
                b1_ref, g2_ref, b2_ref, o_ref, oa_ref, ob_ref):
    psum = [agg_ref[0, 0, t] + agg_ref[0, 1, t] for t in range(6)]
    psum_b = [agg_ref[1, 0, t] + agg_ref[1, 1, t] for t in range(6)]
    for blk in range(2):
        c0 = blk * HD
        x = jnp.concatenate([fa_ref[blk], fb_ref[blk]], axis=-1)
        acc = lax.dot_general(x, wc_ref[...], _DN_T,
                              preferred_element_type=jnp.float32)
        for t in range(6):
            a = jnp.concatenate([psum[t][:, c0:c0 + HD],
                                 psum_b[t][:, c0:c0 + HD]], axis=-1)
            acc = acc + lax.dot_general(a, we_ref[t], _DN_T,
                                        preferred_element_type=jnp.float32)
        mu = jnp.mean(acc, axis=-1, keepdims=True)
        var = jnp.mean((acc - mu) ** 2, axis=-1, keepdims=True)
        h = (acc - mu) * lax.rsqrt(var + 1e-5) * g1_ref[...] + b1_ref[...]
        h = jnp.maximum(h, 0.0)
        y = lax.dot_general(h, wc2_ref[...], _DN_T,
                            preferred_element_type=jnp.float32)
        mu2 = jnp.mean(y, axis=-1, keepdims=True)
        var2 = jnp.mean((y - mu2) ** 2, axis=-1, keepdims=True)
        y = (y - mu2) * lax.rsqrt(var2 + 1e-5) * g2_ref[...] + b2_ref[...]
        o = jnp.maximum(y + x, 0.0)
        o_ref[blk] = o
        oa_ref[blk] = o[:, :HD]
        ob_ref[blk] = o[:, HD:]


def _layer(fa, fb, aggs, wc, we, wc2, g1, b1, g2, b2):
    return pl.pallas_call(
        _layer_body,
        grid=(HN // RB,),
        in_specs=[
            pl.BlockSpec((2, RB, HD), lambda i: (0, i, 0)),
            pl.BlockSpec((2, RB, HD), lambda i: (0, i, 0)),
            pl.BlockSpec((2, 2, 6, RB, D), lambda i: (0, 0, 0, i, 0)),
            pl.BlockSpec((D, D), lambda i: (0, 0)),
            pl.BlockSpec((6, D, D), lambda i: (0, 0, 0)),
            pl.BlockSpec((D, D), lambda i: (0, 0)),
            pl.BlockSpec((1, D), lambda i: (0, 0)),
            pl.BlockSpec((1, D), lambda i: (0, 0)),
            pl.BlockSpec((1, D), lambda i: (0, 0)),
            pl.BlockSpec((1, D), lambda i: (0, 0)),
        ],
        out_specs=[
            pl.BlockSpec((2, RB, D), lambda i: (0, i, 0)),
            pl.BlockSpec((2, RB, HD), lambda i: (0, i, 0)),
            pl.BlockSpec((2, RB, HD), lambda i: (0, i, 0)),
        ],
        out_shape=[
            jax.ShapeDtypeStruct((2, HN, D), jnp.float32),
            jax.ShapeDtypeStruct((2, HN, HD), jnp.float32),
            jax.ShapeDtypeStruct((2, HN, HD), jnp.float32),
        ],
    )(fa, fb, aggs, wc, we, wc2, g1, b1, g2, b2)


def _pack_indices(u_list, v_list):
    us, vs = [], []
    for (u, v, cpt) in zip(u_list, v_list, CPT):
        ep = cpt * NTILES * CHUNK
        e = u.shape[0]
        fill = jnp.arange(ep, dtype=jnp.int32)
        uperm = 2 * (u % HN) + u // HN
        up = (DUMMY + fill % (NROWS - N)).at[:e].set(uperm)
        vp = (fill % N).at[:e].set(v)
        us.append(up.reshape(cpt, NTILES, CHUNK).swapaxes(0, 1))
        vs.append(vp.reshape(cpt, NTILES, CHUNK).swapaxes(0, 1))
    return jnp.concatenate(us, axis=1), jnp.concatenate(vs, axis=1)


def kernel(feats, pre0_u, pre0_v, pre1_u, pre1_v, suc0_u, suc0_v, suc1_u,
           suc1_v, left_u, left_v, right_u, right_v, W_q, W_k, W_v, W_ctr,
           W_pre0, W_pre1, W_suc0, W_suc1, W_left, W_right, W_ctr2, gn1_g,
           gn1_b, gn2_g, gn2_b):
    feats_p = jnp.pad(feats.reshape(B, L, D), ((0, 0), (0, LP - L), (0, 0)))
    att_a, att_b = _attention(feats_p, W_q, W_k, W_v)
    fa = att_a[:, :L].reshape(2, HN, HD)
    fb = att_b[:, :L].reshape(2, HN, HD)

    u_idx, v_idx = _pack_indices(
        [pre0_u, pre1_u, suc0_u, suc1_u, left_u, right_u],
        [pre0_v, pre1_v, suc0_v, suc1_v, left_v, right_v],
    )
    We = jnp.stack([W_pre0, W_pre1, W_suc0, W_suc1, W_left, W_right], axis=1)
    Z = jnp.zeros_like(We[:, 0])
    We = We - jnp.stack([We[:, 1], Z, We[:, 3], Z, We[:, 5], Z], axis=1)
    feat = None
    for i in range(NUM_LAYERS):
        aggs = _sc_agg(fa.reshape(N, HD), fb.reshape(N, HD), u_idx, v_idx)
        aggs = jnp.reshape(aggs, (2, 2, 6, NROWS // 2, D))
        feat, fa, fb = _layer(fa, fb, aggs, W_ctr[i], We[i], W_ctr2[i],
                              gn1_g[i][None], gn1_b[i][None],
                              gn2_g[i][None], gn2_b[i][None])
    return feat.reshape(N, D)


# speedup vs baseline: 1.0512x; 1.0512x over previous
import functools

import jax
import jax.numpy as jnp
import numpy as np
from jax import lax
from jax.experimental import pallas as pl
from jax.experimental.pallas import tpu as pltpu
from jax.experimental.pallas import tpu_sc as plsc

N = 10000
D = 128
HD = 64
B = 16
L = 625
LP = 640
NUM_LAYERS = 4

NTILES = 32
CHUNK = 128
NROWS = 10240
DUMMY = N
SUBROWS = NROWS // 16
ECOUNTS = (80000, 80000, 80000, 80000, 20000, 20000)
CPT = tuple(-(-e // (NTILES * CHUNK)) for e in ECOUNTS)
OFF = tuple(int(np.cumsum((0,) + CPT)[i]) for i in range(6))
TOT = sum(CPT)

_DN_T = (((1,), (1,)), ((), ()))
_DN_N = (((1,), (0,)), ((), ()))


def _attn_body(x_ref, wq_ref, wk_ref, wv_ref, oa_ref, ob_ref):
    x = x_ref[0]
    q = lax.dot_general(x, wq_ref[...], _DN_T, preferred_element_type=jnp.float32)
    k = lax.dot_general(x, wk_ref[...], _DN_T, preferred_element_type=jnp.float32)
    v = lax.dot_general(x, wv_ref[...], _DN_T, preferred_element_type=jnp.float32)
    e = lax.dot_general(q, k, _DN_T, preferred_element_type=jnp.float32)
    e = e * jnp.float32(1.0 / np.sqrt(D))
    col = lax.broadcasted_iota(jnp.int32, (LP, LP), 1)
    e = jnp.where(col < L, e, jnp.float32(-1e30))
    m = jnp.max(e, axis=-1, keepdims=True)
    p = jnp.exp(e - m)
    p = p / jnp.sum(p, axis=-1, keepdims=True)
    att = lax.dot_general(p, v, _DN_N, preferred_element_type=jnp.float32)
    o = x + att
    oa_ref[0] = o[:, :HD]
    ob_ref[0] = o[:, HD:]


def _attention(feats_p, wq, wk, wv):
    return pl.pallas_call(
        _attn_body,
        grid=(B,),
        in_specs=[
            pl.BlockSpec((1, LP, D), lambda b: (b, 0, 0)),
            pl.BlockSpec((D, D), lambda b: (0, 0)),
            pl.BlockSpec((D, D), lambda b: (0, 0)),
            pl.BlockSpec((D, D), lambda b: (0, 0)),
        ],
        out_specs=[
            pl.BlockSpec((1, LP, HD), lambda b: (b, 0, 0)),
            pl.BlockSpec((1, LP, HD), lambda b: (b, 0, 0)),
        ],
        out_shape=[
            jax.ShapeDtypeStruct((B, LP, HD), jnp.float32),
            jax.ShapeDtypeStruct((B, LP, HD), jnp.float32),
        ],
    )(feats_p, wq, wk, wv)


_MESH = plsc.VectorSubcoreMesh(core_axis_name="c", subcore_axis_name="s")


@functools.partial(
    pl.kernel,
    out_type=jax.ShapeDtypeStruct((2, 2, 6, NROWS, HD), jnp.float32),
    mesh=_MESH,
    compiler_params=pltpu.CompilerParams(use_tc_tiling_on_sc=False),
    scratch_types=[
        pltpu.VMEM((TOT, CHUNK), jnp.int32),
        pltpu.VMEM((TOT, CHUNK), jnp.int32),
        pltpu.VMEM((CHUNK, HD), jnp.float32),
        pltpu.VMEM((CHUNK, HD), jnp.float32),
        pltpu.VMEM((CHUNK, HD), jnp.float32),
        pltpu.VMEM((CHUNK, HD), jnp.float32),
        pltpu.VMEM((CHUNK, HD), jnp.float32),
        pltpu.SemaphoreType.DMA,
        pltpu.SemaphoreType.DMA,
        pltpu.SemaphoreType.DMA,
        pltpu.SemaphoreType.DMA,
        pltpu.SemaphoreType.DMA,
        pltpu.SemaphoreType.DMA,
        pltpu.SemaphoreType.DMA,
        pltpu.SemaphoreType.DMA,
        pltpu.VMEM_SHARED((NROWS, HD), jnp.float32),
    ],
)
def _sc_agg(fa_hbm, fb_hbm, u_hbm, v_hbm, out_hbm, u_v, v_v, b0, b1, b2, b3,
            z_v, g0, g1, g2, g3, s0, s1, s2, s3, agg_sh):
    bufs = (b0, b1, b2, b3)
    gs = (g0, g1, g2, g3)
    ss = (s0, s1, s2, s3)
    cid = lax.axis_index("c")
    sid = lax.axis_index("s")
    wid = cid * 16 + sid
    pltpu.sync_copy(u_hbm.at[wid], u_v)
    pltpu.sync_copy(v_hbm.at[wid], v_v)

    def zero_body(i, _):
        z_v[i // 4, pl.ds((i % 4) * 16, 16)] = jnp.zeros((16,), jnp.float32)
        return 0

    lax.fori_loop(0, CHUNK * 4, zero_body, 0)
    row0 = sid * SUBROWS

    def prologue(f_hbm, t):
        for k in range(4):
            pltpu.async_copy(f_hbm.at[v_v.at[OFF[t] + k]], bufs[k], gs[k])

    for d, f_hbm in enumerate((fa_hbm, fb_hbm)):
        if d == 0:
            prologue(f_hbm, 0)
        for t in range(6):
            if t % 2 == 0:
                for j in range(SUBROWS // CHUNK):
                    pltpu.sync_copy(
                        z_v, agg_sh.at[pl.ds(row0 + j * CHUNK, CHUNK)])
                plsc.subcore_barrier()
            lo = OFF[t]
            nq = CPT[t] // 4
            rem = CPT[t] % 4

            def quad_body(i, _):
                base = lo + 4 * i
                for k in range(4):
                    c = base + k
                    pltpu.make_async_copy(
                        f_hbm.at[v_v.at[c]], bufs[k], gs[k]).wait()
                    pltpu.async_copy(
                        bufs[k], agg_sh.at[u_v.at[c]], ss[k], add=True)
                for k in range(4):
                    c = base + k
                    pltpu.make_async_copy(
                        bufs[k], agg_sh.at[u_v.at[c]], ss[k]).wait()

                    @pl.when(i < nq - 1)
                    def _(k=k, c=c):
                        pltpu.async_copy(
                            f_hbm.at[v_v.at[c + 4]], bufs[k], gs[k])
                return 0

            lax.fori_loop(0, nq, quad_body, 0)
            for k in range(rem):
                c = lo + 4 * nq + k
                pltpu.async_copy(f_hbm.at[v_v.at[c]], bufs[k], gs[k])
            for k in range(rem):
                c = lo + 4 * nq + k
                pltpu.make_async_copy(
                    f_hbm.at[v_v.at[c]], bufs[k], gs[k]).wait()
                pltpu.async_copy(
                    bufs[k], agg_sh.at[u_v.at[c]], ss[k], add=True)
            for k in range(rem):
                c = lo + 4 * nq + k
                pltpu.make_async_copy(
                    bufs[k], agg_sh.at[u_v.at[c]], ss[k]).wait()
            if t < 5:
                prologue(f_hbm, t + 1)
            elif d == 0:
                prologue(fb_hbm, 0)
            plsc.subcore_barrier()
            pltpu.sync_copy(
                agg_sh.at[pl.ds(row0, SUBROWS)],
                out_hbm.at[d, cid, t, pl.ds(row0, SUBROWS)],
            )
            plsc.subcore_barrier()


HN = N // 2
RB = 1000


def _layer_body(fa_ref, fb_ref, agg_ref, wc_ref, we_ref, wc2_ref, g1_ref,
                b1_ref, g2_ref, b2_ref, o_ref, oa_ref, ob_ref):
    psum = [agg_ref[0, 0, t] + agg_ref[0, 1, t] for t in range(6)]
    psum_b = [agg_ref[1, 0, t] + agg_ref[1, 1, t] for t in range(6)]
    for blk in range(2):
        c0 = blk * HD
        x = jnp.concatenate([fa_ref[blk], fb_ref[blk]], axis=-1)
        acc = lax.dot_general(x, wc_ref[...], _DN_T,
                              preferred_element_type=jnp.float32)
        for t in range(6):
            a = jnp.concatenate([psum[t][:, c0:c0 + HD],
                                 psum_b[t][:, c0:c0 + HD]], axis=-1)
            acc = acc + lax.dot_general(a, we_ref[t], _DN_T,
                                        preferred_element_type=jnp.float32)
        mu = jnp.mean(acc, axis=-1, keepdims=True)
        var = jnp.mean((acc - mu) ** 2, axis=-1, keepdims=True)
        h = (acc - mu) * lax.rsqrt(var + 1e-5) * g1_ref[...] + b1_ref[...]
        h = jnp.maximum(h, 0.0)
        y = lax.dot_general(h, wc2_ref[...], _DN_T,
                            preferred_element_type=jnp.float32)
        mu2 = jnp.mean(y, axis=-1, keepdims=True)
        var2 = jnp.mean((y - mu2) ** 2, axis=-1, keepdims=True)
        y = (y - mu2) * lax.rsqrt(var2 + 1e-5) * g2_ref[...] + b2_ref[...]
        o = jnp.maximum(y + x, 0.0)
        o_ref[blk] = o
        oa_ref[blk] = o[:, :HD]
        ob_ref[blk] = o[:, HD:]


def _layer(fa, fb, aggs, wc, we, wc2, g1, b1, g2, b2):
    return pl.pallas_call(
        _layer_body,
        grid=(HN // RB,),
        in_specs=[
            pl.BlockSpec((2, RB, HD), lambda i: (0, i, 0)),
            pl.BlockSpec((2, RB, HD), lambda i: (0, i, 0)),
            pl.BlockSpec((2, 2, 6, RB, D), lambda i: (0, 0, 0, i, 0)),
            pl.BlockSpec((D, D), lambda i: (0, 0)),
            pl.BlockSpec((6, D, D), lambda i: (0, 0, 0)),
            pl.BlockSpec((D, D), lambda i: (0, 0)),
            pl.BlockSpec((1, D), lambda i: (0, 0)),
            pl.BlockSpec((1, D), lambda i: (0, 0)),
            pl.BlockSpec((1, D), lambda i: (0, 0)),
            pl.BlockSpec((1, D), lambda i: (0, 0)),
        ],
        out_specs=[
            pl.BlockSpec((2, RB, D), lambda i: (0, i, 0)),
            pl.BlockSpec((2, RB, HD), lambda i: (0, i, 0)),
            pl.BlockSpec((2, RB, HD), lambda i: (0, i, 0)),
        ],
        out_shape=[
            jax.ShapeDtypeStruct((2, HN, D), jnp.float32),
            jax.ShapeDtypeStruct((2, HN, HD), jnp.float32),
            jax.ShapeDtypeStruct((2, HN, HD), jnp.float32),
        ],
    )(fa, fb, aggs, wc, we, wc2, g1, b1, g2, b2)


def _pack_indices(u_list, v_list):
    us, vs = [], []
    for (u, v, cpt) in zip(u_list, v_list, CPT):
        ep = cpt * NTILES * CHUNK
        e = u.shape[0]
        fill = jnp.arange(ep, dtype=jnp.int32)
        uperm = 2 * (u % HN) + u // HN
        up = (DUMMY + fill % (NROWS - N)).at[:e].set(uperm)
        vp = (fill % N).at[:e].set(v)
        us.append(up.reshape(cpt, NTILES, CHUNK).swapaxes(0, 1))
        vs.append(vp.reshape(cpt, NTILES, CHUNK).swapaxes(0, 1))
    return jnp.concatenate(us, axis=1), jnp.concatenate(vs, axis=1)


def kernel(feats, pre0_u, pre0_v, pre1_u, pre1_v, suc0_u, suc0_v, suc1_u,
           suc1_v, left_u, left_v, right_u, right_v, W_q, W_k, W_v, W_ctr,
           W_pre0, W_pre1, W_suc0, W_suc1, W_left, W_right, W_ctr2, gn1_g,
           gn1_b, gn2_g, gn2_b):
    feats_p = jnp.pad(feats.reshape(B, L, D), ((0, 0), (0, LP - L), (0, 0)))
    att_a, att_b = _attention(feats_p, W_q, W_k, W_v)
    fa = att_a[:, :L].reshape(2, HN, HD)
    fb = att_b[:, :L].reshape(2, HN, HD)

    u_idx, v_idx = _pack_indices(
        [pre0_u, pre1_u, suc0_u, suc1_u, left_u, right_u],
        [pre0_v, pre1_v, suc0_v, suc1_v, left_v, right_v],
    )
    We = jnp.stack([W_pre0, W_pre1, W_suc0, W_suc1, W_left, W_right], axis=1)
    Z = jnp.zeros_like(We[:, 0])
    We = We - jnp.stack([We[:, 1], Z, We[:, 3], Z, We[:, 5], Z], axis=1)
    feat = None
    for i in range(NUM_LAYERS):
        aggs = _sc_agg(fa.reshape(N, HD), fb.reshape(N, HD), u_idx, v_idx)
        aggs = jnp.reshape(aggs, (2, 2, 6, NROWS // 2, D))
        feat, fa, fb = _layer(fa, fb, aggs, W_ctr[i], We[i], W_ctr2[i],
                              gn1_g[i][None], gn1_b[i][None],
                              gn2_g[i][None], gn2_b[i][None])
    return feat.reshape(N, D)
